# baseline (speedup 1.0000x reference)
import math

import jax
import jax.numpy as jnp
from jax import lax
from jax.experimental import pallas as pl
from jax.experimental.pallas import tpu as pltpu
from jax.experimental.pallas import tpu_sc as plsc

B = 1024
HB = B // 2
KD = 64
VD = 128
NS = 100000
K = 32

BLK = 1024
NBLK = 98
NS_PAD = NBLK * BLK
TILE = 128
NT = NS_PAD // TILE
TPB = BLK // TILE

NEG = -3.0e38
F32 = jnp.float32
I32 = jnp.int32

NW = 32
RPW = HB // NW
CMAX = K * TILE
CCHUNKS = CMAX // 16
COMP = CMAX + 32


def _proj_body(query_ref, w_ref, b_ref, q_ref):
    q = lax.dot_general(query_ref[...], w_ref[...],
                        (((1,), (1,)), ((), ())),
                        preferred_element_type=F32)
    q_ref[...] = q + b_ref[...]


def _proj(query, W, b):
    return pl.pallas_call(
        _proj_body,
        out_shape=jax.ShapeDtypeStruct((B, KD), F32),
    )(query, W, b.reshape(1, KD))


def _scores_body(q_ref, k_ref, sc_ref, tm_ref):
    i = pl.program_id(0)
    s = lax.dot_general(q_ref[...], k_ref[...],
                        (((1,), (0,)), ((), ())),
                        preferred_element_type=F32) * (1.0 / math.sqrt(KD))
    col = i * BLK + lax.broadcasted_iota(I32, (HB, BLK), 1)
    s = jnp.where(col < NS, s, NEG)
    s3 = s.reshape(HB, TPB, TILE)
    sc_ref[...] = s3
    tm_ref[...] = jnp.max(s3, axis=-1).reshape(1, HB, TPB)


def _scores(q_h, keys_t):
    return pl.pallas_call(
        _scores_body,
        grid=(NBLK,),
        in_specs=[
            pl.BlockSpec((HB, KD), lambda i: (0, 0)),
            pl.BlockSpec((KD, BLK), lambda i: (0, i)),
        ],
        out_specs=[
            pl.BlockSpec((HB, TPB, TILE), lambda i: (0, i, 0)),
            pl.BlockSpec((1, HB, TPB), lambda i: (i, 0, 0)),
        ],
        out_shape=[
            jax.ShapeDtypeStruct((HB, NT, TILE), F32),
            jax.ShapeDtypeStruct((NBLK, HB, TPB), F32),
        ],
        compiler_params=pltpu.CompilerParams(
            vmem_limit_bytes=100 * 1024 * 1024),
    )(q_h, keys_t)


ROWBLK = 128


def _tiles_body(tm_ref, tids_ref, tau_ref):
    tm = tm_ref[...]
    cols = lax.broadcasted_iota(I32, (ROWBLK, NT), 1)
    ids = []
    m = None
    for _ in range(K):
        m = jnp.max(tm, axis=1, keepdims=True)
        eq = tm == m
        pos = jnp.min(jnp.where(eq, cols, 10_000_000), axis=1, keepdims=True)
        ids.append(pos)
        tm = jnp.where(cols == pos, NEG, tm)
    tids_ref[...] = jnp.concatenate(ids, axis=1)
    tau_ref[...] = jnp.broadcast_to(m, (ROWBLK, 16))


def _tiles(tilemax_h):
    return pl.pallas_call(
        _tiles_body,
        grid=(HB // ROWBLK,),
        in_specs=[pl.BlockSpec((ROWBLK, NT), lambda i: (i, 0))],
        out_specs=[
            pl.BlockSpec((ROWBLK, K), lambda i: (i, 0)),
            pl.BlockSpec((ROWBLK, 16), lambda i: (i, 0)),
        ],
        out_shape=[
            jax.ShapeDtypeStruct((HB, K), I32),
            jax.ShapeDtypeStruct((HB, 16), F32),
        ],
        compiler_params=pltpu.CompilerParams(
            vmem_limit_bytes=100 * 1024 * 1024),
    )(tilemax_h)


def _splat(x):
    return lax.broadcast_in_dim(x, (16,), ())


def _topk_sc_body(sc_ref, tids_ref, tau_ref, out_ref,
                  idx_v, rowbuf_v, comp_v, tau_v,
                  vals_v, cnts_v, out_v, sem):
    wid = lax.axis_index("s") * 2 + lax.axis_index("c")
    r0 = wid * RPW

    pltpu.sync_copy(tids_ref.at[pl.ds(r0 * K, RPW * K)], idx_v)
    pltpu.sync_copy(tau_ref.at[pl.ds(r0 * 16, RPW * 16)], tau_v)

    def adj(c, carry):
        sl = pl.ds(c * 16, 16)
        base = (r0 + c // 2) * NT
        idx_v[sl] = idx_v[sl] + base
        return carry
    lax.fori_loop(0, RPW * 2, adj, 0)

    iota16 = lax.iota(I32, 16)

    def row_body(i, carry):
        pltpu.async_copy(sc_ref.at[idx_v.at[pl.ds(i * K, K)]],
                         rowbuf_v, sem).wait()

        def clr(k, c):
            comp_v[pl.ds(k * 16, 16)] = jnp.full((16,), NEG, F32)
            return c
        lax.fori_loop(0, COMP // 16, clr, 0)
        vals_v[pl.ds(0, 16)] = jnp.full((16,), NEG, F32)
        vals_v[pl.ds(16, 16)] = jnp.full((16,), NEG, F32)
        cnts_v[pl.ds(0, 16)] = jnp.zeros((16,), F32)
        cnts_v[pl.ds(16, 16)] = jnp.zeros((16,), F32)

        tau_s = jnp.max(tau_v[pl.ds(i * 16, 16)])

        cb = jnp.zeros((16,), I32)
        for c in range(CCHUNKS):
            v = rowbuf_v[c // (TILE // 16), pl.ds((c % (TILE // 16)) * 16, 16)]
            msk = v >= tau_s
            pos = cb + plsc.cumsum(jnp.where(msk, 1, 0).astype(I32)) - 1
            plsc.store_scatter(comp_v, [pos], v, mask=msk)
            cb = cb + plsc.all_reduce_population_count(msk)
        c_s = jnp.max(cb)
        nc = (c_s + 15) // 16

        def ext(j, car):
            taken, t32, mbest = car

            def mx(k, a):
                return jnp.maximum(a, comp_v[pl.ds(k * 16, 16)])
            acc = lax.fori_loop(0, nc, mx, jnp.full((16,), NEG, F32))
            vj = jnp.max(acc)
            act = taken < K
            act_v = lax.broadcast_in_dim(act, (16,), ())

            def cnt(k, cv):
                sl = pl.ds(k * 16, 16)
                ch = comp_v[sl]
                eq = (ch == vj) & act_v
                comp_v[sl] = jnp.where(eq, NEG, ch)
                return cv + plsc.all_reduce_population_count(eq)
            cv = lax.fori_loop(0, nc, cnt, jnp.zeros((16,), I32))
            cj = jnp.max(cv)

            rec = (iota16 == 0) & act_v
            plsc.store_scatter(vals_v, [_splat(j)], _splat(vj), mask=rec)
            plsc.store_scatter(cnts_v, [_splat(j)],
                               _splat(cj.astype(F32)), mask=rec)

            taken = taken + jnp.where(act, cj, 0)
            t32 = jnp.where(act, vj, t32)
            mbest = jnp.where(j == 0, vj, mbest)
            return (taken, t32, mbest)

        taken, t32_s, m_s = lax.fori_loop(
            0, K, ext, (jnp.int32(0), jnp.float32(NEG), jnp.float32(NEG)))

        zv = jnp.zeros((16,), F32)
        for h in range(2):
            va = vals_v[pl.ds(h * 16, 16)]
            ca = cnts_v[pl.ds(h * 16, 16)]
            e = jnp.where(va > -1.0e37, jnp.exp(va - m_s) * ca, 0.0)
            zv = zv + e
        z_s = jnp.sum(zv)

        out_v[...] = jnp.where(
            iota16 == 0, t32_s,
            jnp.where(iota16 == 1, m_s,
                      jnp.where(iota16 == 2, z_s, 0.0)))
        pltpu.sync_copy(out_v, out_ref.at[pl.ds((r0 + i) * 16, 16)])
        return carry

    lax.fori_loop(0, RPW, row_body, 0)


def _topk_sc(scores_flat, tids_flat, tau_flat):
    mesh = plsc.VectorSubcoreMesh(core_axis_name="c", subcore_axis_name="s")
    f = pl.kernel(
        _topk_sc_body,
        out_type=jax.ShapeDtypeStruct((HB * 16,), F32),
        mesh=mesh,
        scratch_types=[
            pltpu.VMEM((RPW * K,), I32),
            pltpu.VMEM((K, TILE), F32),
            pltpu.VMEM((COMP,), F32),
            pltpu.VMEM((RPW * 16,), F32),
            pltpu.VMEM((K,), F32),
            pltpu.VMEM((K,), F32),
            pltpu.VMEM((16,), F32),
            pltpu.SemaphoreType.DMA,
        ],
        compiler_params=pltpu.CompilerParams(needs_layout_passes=False),
    )
    return f(scores_flat, tids_flat, tau_flat)


def _retr_body(q_ref, k_ref, v_ref, st_ref, w_ref, r_ref):
    i = pl.program_id(0)
    st = lax.dot_general(k_ref[...], q_ref[...],
                         (((0,), (1,)), ((), ())),
                         preferred_element_type=F32) * (1.0 / math.sqrt(KD))
    t32 = st_ref[0:1, :]
    m = st_ref[1:2, :]
    invz = 1.0 / st_ref[2:3, :]
    row = i * BLK + lax.broadcasted_iota(I32, (BLK, 1), 0)
    w = jnp.where((st >= t32) & (row < NS), jnp.exp(st - m) * invz, 0.0)
    w_ref[...] = w
    v = jnp.where(row < NS, v_ref[...], 0.0)
    pv = lax.dot_general(w, v, (((0,), (0,)), ((), ())),
                         preferred_element_type=F32)

    @pl.when(i == 0)
    def _():
        r_ref[...] = jnp.zeros_like(r_ref)

    r_ref[...] += pv


def _retr_body_h1(wbuf_ref, q_ref, k_ref, v_ref, st_ref, w_ref, r_ref):
    del wbuf_ref
    _retr_body(q_ref, k_ref, v_ref, st_ref, w_ref, r_ref)


def _retrieve_h0(q_h, keys_t, values, stats_t):
    return pl.pallas_call(
        _retr_body,
        grid=(NBLK,),
        in_specs=[
            pl.BlockSpec((HB, KD), lambda i: (0, 0)),
            pl.BlockSpec((KD, BLK), lambda i: (0, i)),
            pl.BlockSpec((BLK, VD), lambda i: (i, 0)),
            pl.BlockSpec((16, HB), lambda i: (0, 0)),
        ],
        out_specs=[
            pl.BlockSpec((BLK, HB), lambda i: (i, 0)),
            pl.BlockSpec((HB, VD), lambda i: (0, 0)),
        ],
        out_shape=[
            jax.ShapeDtypeStruct((NS, B), F32),
            jax.ShapeDtypeStruct((HB, VD), F32),
        ],
        compiler_params=pltpu.CompilerParams(
            vmem_limit_bytes=100 * 1024 * 1024),
    )(q_h, keys_t, values, stats_t)


def _retrieve_h1(wbuf, q_h, keys_t, values, stats_t):
    return pl.pallas_call(
        _retr_body_h1,
        grid=(NBLK,),
        in_specs=[
            pl.BlockSpec(memory_space=pl.ANY),
            pl.BlockSpec((HB, KD), lambda i: (0, 0)),
            pl.BlockSpec((KD, BLK), lambda i: (0, i)),
            pl.BlockSpec((BLK, VD), lambda i: (i, 0)),
            pl.BlockSpec((16, HB), lambda i: (0, 0)),
        ],
        out_specs=[
            pl.BlockSpec((BLK, HB), lambda i: (i, 1)),
            pl.BlockSpec((HB, VD), lambda i: (0, 0)),
        ],
        out_shape=[
            jax.ShapeDtypeStruct((NS, B), F32),
            jax.ShapeDtypeStruct((HB, VD), F32),
        ],
        input_output_aliases={0: 0},
        compiler_params=pltpu.CompilerParams(
            vmem_limit_bytes=100 * 1024 * 1024),
    )(wbuf, q_h, keys_t, values, stats_t)


def _half(q_h, keys_t):
    scores, tilemax = _scores(q_h, keys_t)
    tids, tau = _tiles(tilemax.transpose(1, 0, 2).reshape(HB, NT))
    stats = _topk_sc(scores.reshape(HB * NT, TILE),
                     tids.reshape(-1), tau.reshape(-1))
    return stats.reshape(HB, 16).T


@jax.jit
def _run(query, keys, values, W, b):
    keys_t = keys.T
    q = _proj(query, W, b)
    q0, q1 = q[:HB], q[HB:]
    st0 = _half(q0, keys_t)
    st1 = _half(q1, keys_t)
    wbuf, r0 = _retrieve_h0(q0, keys_t, values, st0)
    weights_t, r1 = _retrieve_h1(wbuf, q1, keys_t, values, st1)
    retrieved = jnp.concatenate([r0, r1], axis=0)
    return retrieved, weights_t.reshape(1, NS, B).transpose(2, 0, 1)


def kernel(query, top_k, keys, values, W, b):
    del top_k
    return _run(query, keys, values, W, b)


# speedup vs baseline: 1.0014x; 1.0014x over previous
import math

import jax
import jax.numpy as jnp
from jax import lax
from jax.experimental import pallas as pl
from jax.experimental.pallas import tpu as pltpu
from jax.experimental.pallas import tpu_sc as plsc

B = 1024
HB = B // 2
KD = 64
VD = 128
NS = 100000
K = 32

BLK = 1024
NBLK = 98
NS_PAD = NBLK * BLK
TILE = 128
NT = NS_PAD // TILE
TPB = BLK // TILE

NEG = -3.0e38
F32 = jnp.float32
I32 = jnp.int32

NW = 32
RPW = HB // NW
CMAX = K * TILE
CCHUNKS = CMAX // 16
COMP = CMAX + 32


def _proj_body(query_ref, w_ref, b_ref, q_ref):
    q = lax.dot_general(query_ref[...], w_ref[...],
                        (((1,), (1,)), ((), ())),
                        preferred_element_type=F32)
    q_ref[...] = q + b_ref[...]


def _proj(query, W, b):
    return pl.pallas_call(
        _proj_body,
        out_shape=jax.ShapeDtypeStruct((B, KD), F32),
    )(query, W, b.reshape(1, KD))


def _scores_body(q_ref, k_ref, sc_ref, tm_ref):
    i = pl.program_id(0)
    s = lax.dot_general(q_ref[...], k_ref[...],
                        (((1,), (0,)), ((), ())),
                        preferred_element_type=F32) * (1.0 / math.sqrt(KD))
    col = i * BLK + lax.broadcasted_iota(I32, (HB, BLK), 1)
    s = jnp.where(col < NS, s, NEG)
    s3 = s.reshape(HB, TPB, TILE)
    sc_ref[...] = s3
    tm_ref[...] = jnp.max(s3, axis=-1).reshape(1, HB, TPB)


def _scores(q_h, keys_t):
    return pl.pallas_call(
        _scores_body,
        grid=(NBLK,),
        in_specs=[
            pl.BlockSpec((HB, KD), lambda i: (0, 0)),
            pl.BlockSpec((KD, BLK), lambda i: (0, i)),
        ],
        out_specs=[
            pl.BlockSpec((HB, TPB, TILE), lambda i: (0, i, 0)),
            pl.BlockSpec((1, HB, TPB), lambda i: (i, 0, 0)),
        ],
        out_shape=[
            jax.ShapeDtypeStruct((HB, NT, TILE), F32),
            jax.ShapeDtypeStruct((NBLK, HB, TPB), F32),
        ],
        compiler_params=pltpu.CompilerParams(
            vmem_limit_bytes=100 * 1024 * 1024),
    )(q_h, keys_t)


ROWBLK = 128


def _tiles_body(tm_ref, tids_ref, tau_ref):
    tm = tm_ref[...]
    cols = lax.broadcasted_iota(I32, (ROWBLK, NT), 1)
    ids = []
    m = None
    for _ in range(K):
        m = jnp.max(tm, axis=1, keepdims=True)
        eq = tm == m
        pos = jnp.min(jnp.where(eq, cols, 10_000_000), axis=1, keepdims=True)
        ids.append(pos)
        tm = jnp.where(cols == pos, NEG, tm)
    tids_ref[...] = jnp.concatenate(ids, axis=1)
    tau_ref[...] = jnp.broadcast_to(m, (ROWBLK, 16))


def _tiles(tilemax_h):
    return pl.pallas_call(
        _tiles_body,
        grid=(HB // ROWBLK,),
        in_specs=[pl.BlockSpec((ROWBLK, NT), lambda i: (i, 0))],
        out_specs=[
            pl.BlockSpec((ROWBLK, K), lambda i: (i, 0)),
            pl.BlockSpec((ROWBLK, 16), lambda i: (i, 0)),
        ],
        out_shape=[
            jax.ShapeDtypeStruct((HB, K), I32),
            jax.ShapeDtypeStruct((HB, 16), F32),
        ],
        compiler_params=pltpu.CompilerParams(
            vmem_limit_bytes=100 * 1024 * 1024),
    )(tilemax_h)


def _splat(x):
    return lax.broadcast_in_dim(x, (16,), ())


def _topk_sc_body(sc_ref, tids_ref, tau_ref, out_ref,
                  idx_v, rowbuf_v, comp_v, tau_v,
                  vals_v, cnts_v, out_v, sem):
    wid = lax.axis_index("s") * 2 + lax.axis_index("c")
    r0 = wid * RPW

    pltpu.sync_copy(tids_ref.at[pl.ds(r0 * K, RPW * K)], idx_v)
    pltpu.sync_copy(tau_ref.at[pl.ds(r0 * 16, RPW * 16)], tau_v)

    def adj(c, carry):
        sl = pl.ds(c * 16, 16)
        base = (r0 + c // 2) * NT
        idx_v[sl] = idx_v[sl] + base
        return carry
    lax.fori_loop(0, RPW * 2, adj, 0)

    iota16 = lax.iota(I32, 16)

    def row_body(i, carry):
        pltpu.async_copy(sc_ref.at[idx_v.at[pl.ds(i * K, K)]],
                         rowbuf_v, sem).wait()

        def clr(k, c):
            comp_v[pl.ds(k * 16, 16)] = jnp.full((16,), NEG, F32)
            return c
        lax.fori_loop(0, COMP // 16, clr, 0)
        vals_v[pl.ds(0, 16)] = jnp.full((16,), NEG, F32)
        vals_v[pl.ds(16, 16)] = jnp.full((16,), NEG, F32)
        cnts_v[pl.ds(0, 16)] = jnp.zeros((16,), F32)
        cnts_v[pl.ds(16, 16)] = jnp.zeros((16,), F32)

        tau_s = jnp.max(tau_v[pl.ds(i * 16, 16)])

        cb = jnp.zeros((16,), I32)
        for c in range(CCHUNKS):
            v = rowbuf_v[c // (TILE // 16), pl.ds((c % (TILE // 16)) * 16, 16)]
            msk = v >= tau_s
            pos = cb + plsc.cumsum(jnp.where(msk, 1, 0).astype(I32)) - 1
            plsc.store_scatter(comp_v, [pos], v, mask=msk)
            cb = cb + plsc.all_reduce_population_count(msk)
        c_s = jnp.max(cb)
        nc = (c_s + 15) // 16

        def ext(j, car):
            taken, t32, mbest = car

            def mx(k, a):
                return jnp.maximum(a, comp_v[pl.ds(k * 16, 16)])
            acc = lax.fori_loop(0, nc, mx, jnp.full((16,), NEG, F32))
            vj = jnp.max(acc)
            act = taken < K
            act_v = lax.broadcast_in_dim(act, (16,), ())

            def cnt(k, cv):
                sl = pl.ds(k * 16, 16)
                ch = comp_v[sl]
                eq = (ch == vj) & act_v
                comp_v[sl] = jnp.where(eq, NEG, ch)
                return cv + plsc.all_reduce_population_count(eq)
            cv = lax.fori_loop(0, nc, cnt, jnp.zeros((16,), I32))
            cj = jnp.max(cv)

            rec = (iota16 == 0) & act_v
            plsc.store_scatter(vals_v, [_splat(j)], _splat(vj), mask=rec)
            plsc.store_scatter(cnts_v, [_splat(j)],
                               _splat(cj.astype(F32)), mask=rec)

            taken = taken + jnp.where(act, cj, 0)
            t32 = jnp.where(act, vj, t32)
            mbest = jnp.where(j == 0, vj, mbest)
            return (taken, t32, mbest)

        taken, t32_s, m_s = lax.fori_loop(
            0, K, ext, (jnp.int32(0), jnp.float32(NEG), jnp.float32(NEG)))

        zv = jnp.zeros((16,), F32)
        for h in range(2):
            va = vals_v[pl.ds(h * 16, 16)]
            ca = cnts_v[pl.ds(h * 16, 16)]
            e = jnp.where(va > -1.0e37, jnp.exp(va - m_s) * ca, 0.0)
            zv = zv + e
        z_s = jnp.sum(zv)

        out_v[...] = jnp.where(
            iota16 == 0, t32_s,
            jnp.where(iota16 == 1, m_s,
                      jnp.where(iota16 == 2, z_s, 0.0)))
        pltpu.sync_copy(out_v, out_ref.at[pl.ds((r0 + i) * 16, 16)])
        return carry

    lax.fori_loop(0, RPW, row_body, 0)


def _topk_sc(scores_flat, tids_flat, tau_flat):
    mesh = plsc.VectorSubcoreMesh(core_axis_name="c", subcore_axis_name="s")
    f = pl.kernel(
        _topk_sc_body,
        out_type=jax.ShapeDtypeStruct((HB * 16,), F32),
        mesh=mesh,
        scratch_types=[
            pltpu.VMEM((RPW * K,), I32),
            pltpu.VMEM((K, TILE), F32),
            pltpu.VMEM((COMP,), F32),
            pltpu.VMEM((RPW * 16,), F32),
            pltpu.VMEM((K,), F32),
            pltpu.VMEM((K,), F32),
            pltpu.VMEM((16,), F32),
            pltpu.SemaphoreType.DMA,
        ],
        compiler_params=pltpu.CompilerParams(needs_layout_passes=False),
        cost_estimate=pl.CostEstimate(
            flops=40_000_000, bytes_accessed=20_000_000,
            transcendentals=20_000),
    )
    return f(scores_flat, tids_flat, tau_flat)


def _retr_body(q_ref, k_ref, v_ref, st_ref, w_ref, r_ref):
    i = pl.program_id(0)
    st = lax.dot_general(k_ref[...], q_ref[...],
                         (((0,), (1,)), ((), ())),
                         preferred_element_type=F32) * (1.0 / math.sqrt(KD))
    t32 = st_ref[0:1, :]
    m = st_ref[1:2, :]
    invz = 1.0 / st_ref[2:3, :]
    row = i * BLK + lax.broadcasted_iota(I32, (BLK, 1), 0)
    w = jnp.where((st >= t32) & (row < NS), jnp.exp(st - m) * invz, 0.0)
    w_ref[...] = w
    v = jnp.where(row < NS, v_ref[...], 0.0)
    pv = lax.dot_general(w, v, (((0,), (0,)), ((), ())),
                         preferred_element_type=F32)

    @pl.when(i == 0)
    def _():
        r_ref[...] = jnp.zeros_like(r_ref)

    r_ref[...] += pv


def _retr_body_h1(wbuf_ref, q_ref, k_ref, v_ref, st_ref, w_ref, r_ref):
    del wbuf_ref
    _retr_body(q_ref, k_ref, v_ref, st_ref, w_ref, r_ref)


def _retrieve_h0(q_h, keys_t, values, stats_t):
    return pl.pallas_call(
        _retr_body,
        grid=(NBLK,),
        in_specs=[
            pl.BlockSpec((HB, KD), lambda i: (0, 0)),
            pl.BlockSpec((KD, BLK), lambda i: (0, i)),
            pl.BlockSpec((BLK, VD), lambda i: (i, 0)),
            pl.BlockSpec((16, HB), lambda i: (0, 0)),
        ],
        out_specs=[
            pl.BlockSpec((BLK, HB), lambda i: (i, 0)),
            pl.BlockSpec((HB, VD), lambda i: (0, 0)),
        ],
        out_shape=[
            jax.ShapeDtypeStruct((NS, B), F32),
            jax.ShapeDtypeStruct((HB, VD), F32),
        ],
        compiler_params=pltpu.CompilerParams(
            vmem_limit_bytes=100 * 1024 * 1024),
    )(q_h, keys_t, values, stats_t)


def _retrieve_h1(wbuf, q_h, keys_t, values, stats_t):
    return pl.pallas_call(
        _retr_body_h1,
        grid=(NBLK,),
        in_specs=[
            pl.BlockSpec(memory_space=pl.ANY),
            pl.BlockSpec((HB, KD), lambda i: (0, 0)),
            pl.BlockSpec((KD, BLK), lambda i: (0, i)),
            pl.BlockSpec((BLK, VD), lambda i: (i, 0)),
            pl.BlockSpec((16, HB), lambda i: (0, 0)),
        ],
        out_specs=[
            pl.BlockSpec((BLK, HB), lambda i: (i, 1)),
            pl.BlockSpec((HB, VD), lambda i: (0, 0)),
        ],
        out_shape=[
            jax.ShapeDtypeStruct((NS, B), F32),
            jax.ShapeDtypeStruct((HB, VD), F32),
        ],
        input_output_aliases={0: 0},
        compiler_params=pltpu.CompilerParams(
            vmem_limit_bytes=100 * 1024 * 1024),
    )(wbuf, q_h, keys_t, values, stats_t)


def _half(q_h, keys_t):
    scores, tilemax = _scores(q_h, keys_t)
    tids, tau = _tiles(tilemax.transpose(1, 0, 2).reshape(HB, NT))
    stats = _topk_sc(scores.reshape(HB * NT, TILE),
                     tids.reshape(-1), tau.reshape(-1))
    return stats.reshape(HB, 16).T


@jax.jit
def _run(query, keys, values, W, b):
    keys_t = keys.T
    q = _proj(query, W, b)
    q0, q1 = q[:HB], q[HB:]
    st0 = _half(q0, keys_t)
    st1 = _half(q1, keys_t)
    wbuf, r0 = _retrieve_h0(q0, keys_t, values, st0)
    weights_t, r1 = _retrieve_h1(wbuf, q1, keys_t, values, st1)
    retrieved = jnp.concatenate([r0, r1], axis=0)
    return retrieved, weights_t.reshape(1, NS, B).transpose(2, 0, 1)


def kernel(query, top_k, keys, values, W, b):
    del top_k
    return _run(query, keys, values, W, b)


# speedup vs baseline: 1.1855x; 1.1838x over previous
import math

import jax
import jax.numpy as jnp
from jax import lax
from jax.experimental import pallas as pl
from jax.experimental.pallas import tpu as pltpu
from jax.experimental.pallas import tpu_sc as plsc

B = 1024
HB = B // 2
KD = 64
VD = 128
NS = 100000
K = 32

BLK = 1024
NBLK = 98
NS_PAD = NBLK * BLK
TILE = 128
NT = NS_PAD // TILE
TPB = BLK // TILE

NEG = -3.0e38
F32 = jnp.float32
I32 = jnp.int32

NW = 32
RPW = HB // NW
CMAX = K * TILE
CCHUNKS = CMAX // 16
COMP = CMAX + 32


def _proj_body(query_ref, w_ref, b_ref, q_ref):
    q = lax.dot_general(query_ref[...], w_ref[...],
                        (((1,), (1,)), ((), ())),
                        preferred_element_type=F32)
    q_ref[...] = q + b_ref[...]


def _proj(query, W, b):
    return pl.pallas_call(
        _proj_body,
        out_shape=jax.ShapeDtypeStruct((B, KD), F32),
    )(query, W, b.reshape(1, KD))


def _scores_body(q_ref, k_ref, sc_ref, tm_ref):
    i = pl.program_id(0)
    s = lax.dot_general(q_ref[...], k_ref[...],
                        (((1,), (0,)), ((), ())),
                        preferred_element_type=F32) * (1.0 / math.sqrt(KD))
    col = i * BLK + lax.broadcasted_iota(I32, (HB, BLK), 1)
    s = jnp.where(col < NS, s, NEG)
    s3 = s.reshape(HB, TPB, TILE)
    sc_ref[...] = s3
    tm_ref[...] = jnp.max(s3, axis=-1).reshape(1, HB, TPB)


def _scores(q_h, keys_t):
    return pl.pallas_call(
        _scores_body,
        grid=(NBLK,),
        in_specs=[
            pl.BlockSpec((HB, KD), lambda i: (0, 0)),
            pl.BlockSpec((KD, BLK), lambda i: (0, i)),
        ],
        out_specs=[
            pl.BlockSpec((HB, TPB, TILE), lambda i: (0, i, 0)),
            pl.BlockSpec((1, HB, TPB), lambda i: (i, 0, 0)),
        ],
        out_shape=[
            jax.ShapeDtypeStruct((HB, NT, TILE), F32),
            jax.ShapeDtypeStruct((NBLK, HB, TPB), F32),
        ],
        compiler_params=pltpu.CompilerParams(
            vmem_limit_bytes=100 * 1024 * 1024),
    )(q_h, keys_t)


ROWBLK = 128


def _tiles_body(tm_ref, tids_ref, tau_ref):
    tm = tm_ref[...]
    cols = lax.broadcasted_iota(I32, (ROWBLK, NT), 1)
    ids = []
    m = None
    for _ in range(K):
        m = jnp.max(tm, axis=1, keepdims=True)
        eq = tm == m
        pos = jnp.min(jnp.where(eq, cols, 10_000_000), axis=1, keepdims=True)
        ids.append(pos)
        tm = jnp.where(cols == pos, NEG, tm)
    tids_ref[...] = jnp.concatenate(ids, axis=1)
    tau_ref[...] = jnp.broadcast_to(m, (ROWBLK, 16))


def _tiles(tilemax_h):
    return pl.pallas_call(
        _tiles_body,
        grid=(HB // ROWBLK,),
        in_specs=[pl.BlockSpec((ROWBLK, NT), lambda i: (i, 0))],
        out_specs=[
            pl.BlockSpec((ROWBLK, K), lambda i: (i, 0)),
            pl.BlockSpec((ROWBLK, 16), lambda i: (i, 0)),
        ],
        out_shape=[
            jax.ShapeDtypeStruct((HB, K), I32),
            jax.ShapeDtypeStruct((HB, 16), F32),
        ],
        compiler_params=pltpu.CompilerParams(
            vmem_limit_bytes=100 * 1024 * 1024),
    )(tilemax_h)


def _splat(x):
    return lax.broadcast_in_dim(x, (16,), ())


def _topk_sc_body(sc_ref, tids_ref, tau_ref, out_ref,
                  idx_v, rowbuf_v, comp_v, tau_v,
                  vals_v, cnts_v, out_v, sem):
    wid = lax.axis_index("s") * 2 + lax.axis_index("c")
    r0 = wid * RPW

    pltpu.sync_copy(tids_ref.at[pl.ds(r0 * K, RPW * K)], idx_v)
    pltpu.sync_copy(tau_ref.at[pl.ds(r0 * 16, RPW * 16)], tau_v)

    def adj(c, carry):
        sl = pl.ds(c * 16, 16)
        base = (r0 + c // 2) * NT
        idx_v[sl] = idx_v[sl] + base
        return carry
    lax.fori_loop(0, RPW * 2, adj, 0)

    iota16 = lax.iota(I32, 16)

    def row_body(i, carry):
        pltpu.async_copy(sc_ref.at[idx_v.at[pl.ds(i * K, K)]],
                         rowbuf_v, sem).wait()

        def clr(k, c):
            comp_v[pl.ds(k * 16, 16)] = jnp.full((16,), NEG, F32)
            return c
        lax.fori_loop(0, COMP // 16, clr, 0)
        vals_v[pl.ds(0, 16)] = jnp.full((16,), NEG, F32)
        vals_v[pl.ds(16, 16)] = jnp.full((16,), NEG, F32)
        cnts_v[pl.ds(0, 16)] = jnp.zeros((16,), F32)
        cnts_v[pl.ds(16, 16)] = jnp.zeros((16,), F32)

        tau_s = jnp.max(tau_v[pl.ds(i * 16, 16)])

        cb = jnp.zeros((16,), I32)
        for c in range(CCHUNKS):
            v = rowbuf_v[c // (TILE // 16), pl.ds((c % (TILE // 16)) * 16, 16)]
            msk = v >= tau_s
            pos = cb + plsc.cumsum(jnp.where(msk, 1, 0).astype(I32)) - 1
            plsc.store_scatter(comp_v, [pos], v, mask=msk)
            cb = cb + plsc.all_reduce_population_count(msk)
        c_s = jnp.max(cb)
        nc = (c_s + 15) // 16

        def ext(j, car):
            taken, t32, mbest = car

            def mx(k, a):
                return jnp.maximum(a, comp_v[pl.ds(k * 16, 16)])
            acc = lax.fori_loop(0, nc, mx, jnp.full((16,), NEG, F32))
            vj = jnp.max(acc)
            act = taken < K
            act_v = lax.broadcast_in_dim(act, (16,), ())

            def cnt(k, cv):
                sl = pl.ds(k * 16, 16)
                ch = comp_v[sl]
                eq = (ch == vj) & act_v
                comp_v[sl] = jnp.where(eq, NEG, ch)
                return cv + plsc.all_reduce_population_count(eq)
            cv = lax.fori_loop(0, nc, cnt, jnp.zeros((16,), I32))
            cj = jnp.max(cv)

            rec = (iota16 == 0) & act_v
            plsc.store_scatter(vals_v, [_splat(j)], _splat(vj), mask=rec)
            plsc.store_scatter(cnts_v, [_splat(j)],
                               _splat(cj.astype(F32)), mask=rec)

            taken = taken + jnp.where(act, cj, 0)
            t32 = jnp.where(act, vj, t32)
            mbest = jnp.where(j == 0, vj, mbest)
            return (taken, t32, mbest)

        taken, t32_s, m_s = lax.fori_loop(
            0, K, ext, (jnp.int32(0), jnp.float32(NEG), jnp.float32(NEG)))

        zv = jnp.zeros((16,), F32)
        for h in range(2):
            va = vals_v[pl.ds(h * 16, 16)]
            ca = cnts_v[pl.ds(h * 16, 16)]
            e = jnp.where(va > -1.0e37, jnp.exp(va - m_s) * ca, 0.0)
            zv = zv + e
        z_s = jnp.sum(zv)

        out_v[...] = jnp.where(
            iota16 == 0, t32_s,
            jnp.where(iota16 == 1, m_s,
                      jnp.where(iota16 == 2, z_s, 0.0)))
        pltpu.sync_copy(out_v, out_ref.at[pl.ds((r0 + i) * 16, 16)])
        return carry

    lax.fori_loop(0, RPW, row_body, 0)


def _topk_sc(scores_flat, tids_flat, tau_flat):
    mesh = plsc.VectorSubcoreMesh(core_axis_name="c", subcore_axis_name="s")
    f = pl.kernel(
        _topk_sc_body,
        out_type=jax.ShapeDtypeStruct((HB * 16,), F32),
        mesh=mesh,
        scratch_types=[
            pltpu.VMEM((RPW * K,), I32),
            pltpu.VMEM((K, TILE), F32),
            pltpu.VMEM((COMP,), F32),
            pltpu.VMEM((RPW * 16,), F32),
            pltpu.VMEM((K,), F32),
            pltpu.VMEM((K,), F32),
            pltpu.VMEM((16,), F32),
            pltpu.SemaphoreType.DMA,
        ],
        compiler_params=pltpu.CompilerParams(needs_layout_passes=False),
        cost_estimate=pl.CostEstimate(
            flops=40_000_000, bytes_accessed=20_000_000,
            transcendentals=20_000),
    )
    return f(scores_flat, tids_flat, tau_flat)


def _retr_body(q_ref, k_ref, v_ref, st_ref, w_ref, r_ref):
    i = pl.program_id(0)
    st = lax.dot_general(k_ref[...], q_ref[...],
                         (((0,), (1,)), ((), ())),
                         preferred_element_type=F32) * (1.0 / math.sqrt(KD))
    t32 = st_ref[0:1, :]
    m = st_ref[1:2, :]
    invz = 1.0 / st_ref[2:3, :]
    row = i * BLK + lax.broadcasted_iota(I32, (BLK, 1), 0)
    w = jnp.where((st >= t32) & (row < NS), jnp.exp(st - m) * invz, 0.0)
    w_ref[...] = w
    v = jnp.where(row < NS, v_ref[...], 0.0)
    pv = lax.dot_general(w, v, (((0,), (0,)), ((), ())),
                         preferred_element_type=F32)

    @pl.when(i == 0)
    def _():
        r_ref[...] = jnp.zeros_like(r_ref)

    r_ref[...] += pv


def _retr_body_h1(wbuf_ref, q_ref, k_ref, v_ref, st_ref, w_ref, r_ref):
    del wbuf_ref
    _retr_body(q_ref, k_ref, v_ref, st_ref, w_ref, r_ref)


def _retrieve_h0(q_h, keys_t, values, stats_t):
    return pl.pallas_call(
        _retr_body,
        grid=(NBLK,),
        in_specs=[
            pl.BlockSpec((HB, KD), lambda i: (0, 0)),
            pl.BlockSpec((KD, BLK), lambda i: (0, i)),
            pl.BlockSpec((BLK, VD), lambda i: (i, 0)),
            pl.BlockSpec((16, HB), lambda i: (0, 0)),
        ],
        out_specs=[
            pl.BlockSpec((BLK, HB), lambda i: (i, 0)),
            pl.BlockSpec((HB, VD), lambda i: (0, 0)),
        ],
        out_shape=[
            jax.ShapeDtypeStruct((NS, B), F32),
            jax.ShapeDtypeStruct((HB, VD), F32),
        ],
        compiler_params=pltpu.CompilerParams(
            vmem_limit_bytes=100 * 1024 * 1024),
    )(q_h, keys_t, values, stats_t)


def _retrieve_h1(wbuf, q_h, keys_t, values, stats_t):
    return pl.pallas_call(
        _retr_body_h1,
        grid=(NBLK,),
        in_specs=[
            pl.BlockSpec(memory_space=pl.ANY),
            pl.BlockSpec((HB, KD), lambda i: (0, 0)),
            pl.BlockSpec((KD, BLK), lambda i: (0, i)),
            pl.BlockSpec((BLK, VD), lambda i: (i, 0)),
            pl.BlockSpec((16, HB), lambda i: (0, 0)),
        ],
        out_specs=[
            pl.BlockSpec((BLK, HB), lambda i: (i, 1)),
            pl.BlockSpec((HB, VD), lambda i: (0, 0)),
        ],
        out_shape=[
            jax.ShapeDtypeStruct((NS, B), F32),
            jax.ShapeDtypeStruct((HB, VD), F32),
        ],
        input_output_aliases={0: 0},
        compiler_params=pltpu.CompilerParams(
            vmem_limit_bytes=100 * 1024 * 1024),
    )(wbuf, q_h, keys_t, values, stats_t)


def _half(q_h, keys_t, token=None):
    scores, tilemax = _scores(q_h, keys_t)
    tids, tau = _tiles(tilemax.transpose(1, 0, 2).reshape(HB, NT))
    sc_in = (scores.reshape(HB * NT, TILE), tids.reshape(-1),
             tau.reshape(-1))
    if token is not None:
        sc_in, _ = lax.optimization_barrier((sc_in, token))
    stats = _topk_sc(*sc_in)
    return stats.reshape(HB, 16).T


@jax.jit
def _run(query, keys, values, W, b):
    keys_t = keys.T
    q = _proj(query, W, b)
    q0, q1 = q[:HB], q[HB:]
    st0 = _half(q0, keys_t)
    st1 = _half(q1, keys_t, token=st0)
    wbuf, r0 = _retrieve_h0(q0, keys_t, values, st0)
    weights_t, r1 = _retrieve_h1(wbuf, q1, keys_t, values, st1)
    retrieved = jnp.concatenate([r0, r1], axis=0)
    return retrieved, weights_t.reshape(1, NS, B).transpose(2, 0, 1)


def kernel(query, top_k, keys, values, W, b):
    del top_k
    return _run(query, keys, values, W, b)
